# baseline (speedup 1.0000x reference)
import functools

import jax
import jax.numpy as jnp
from jax import lax
from jax.experimental import pallas as pl
from jax.experimental.pallas import tpu as pltpu
from jax.experimental.pallas import tpu_sc as plsc

N = 50000
E = 800000
NC = 2
NS = 16
NW = NC * NS
RNG = 1568
NPAD = RNG * NW

CH = 2000
FL = 2048
CAP = E + 2 * FL

KB = 256
E_PER_TILE = E // NS
K3 = 2000
ZR = 196
NPD = NS * ZR * 16

_MESH = plsc.VectorSubcoreMesh(core_axis_name="c", subcore_axis_name="s")


def _wid():
    return lax.axis_index("s") * NC + lax.axis_index("c")


def _partition_body(edge_hbm, srcl_hbm, dstl_hbm, cnt_hbm,
                    sbuf, dbuf, stg_s, stg_d, cnt_v):
    w = _wid()
    lo = w * RNG
    hi = lo + RNG

    RMASK = 2 * FL - 1

    def chunk(ci, carry):
        fill, oo = carry
        off = pl.multiple_of(ci * CH, 8)
        pltpu.sync_copy(edge_hbm.at[pl.ds(off, CH)], sbuf)
        pltpu.sync_copy(edge_hbm.at[pl.ds(pl.multiple_of(E + off, 8), CH)],
                        dbuf)

        def vec(vi, fill):
            base = fill
            for k in range(25):
                b = pl.multiple_of(vi * 400 + k * 16, 16)
                s16 = sbuf[pl.ds(b, 16)]
                d16 = dbuf[pl.ds(b, 16)]
                m = (d16 >= lo) & (d16 < hi)
                mi = m.astype(jnp.int32)
                cs = plsc.cumsum(mi)
                dest = (base + cs - mi) & RMASK
                plsc.store_scatter(stg_s, [dest], s16, mask=m)
                plsc.store_scatter(stg_d, [dest], d16 - lo, mask=m)
                base = base + cs[15]
            return base

        fill = lax.fori_loop(0, CH // 400, vec, fill)

        def do_flush(args):
            fill, oo = args
            fp = pl.multiple_of((oo & RMASK), FL)
            foff = pl.multiple_of(w * CAP + oo, 8)
            pltpu.sync_copy(stg_s.at[pl.ds(fp, FL)],
                            srcl_hbm.at[pl.ds(foff, FL)])
            pltpu.sync_copy(stg_d.at[pl.ds(fp, FL)],
                            dstl_hbm.at[pl.ds(foff, FL)])
            return fill, oo + FL

        return lax.cond(fill - oo >= FL, do_flush, lambda a: a, (fill, oo))

    fill, oo = lax.fori_loop(0, E // CH, chunk, (0, 0))
    fp = pl.multiple_of((oo & RMASK), FL)
    foff = pl.multiple_of(w * CAP + oo, 8)
    pltpu.sync_copy(stg_s.at[pl.ds(fp, FL)], srcl_hbm.at[pl.ds(foff, FL)])
    pltpu.sync_copy(stg_d.at[pl.ds(fp, FL)], dstl_hbm.at[pl.ds(foff, FL)])
    cnt_v[pl.ds(0, 16)] = jnp.full((16,), fill, jnp.int32)
    pltpu.sync_copy(cnt_v, cnt_hbm.at[pl.ds(pl.multiple_of(w * 16, 16), 16)])


def _partition(edge_index):
    f = pl.kernel(
        _partition_body,
        compiler_params=pltpu.CompilerParams(needs_layout_passes=False,
                                             use_tc_tiling_on_sc=False),
        out_type=[
            jax.ShapeDtypeStruct((NW * CAP,), jnp.int32),
            jax.ShapeDtypeStruct((NW * CAP,), jnp.int32),
            jax.ShapeDtypeStruct((NW * 16,), jnp.int32),
        ],
        mesh=_MESH,
        scratch_types=[
            pltpu.VMEM((CH,), jnp.int32),
            pltpu.VMEM((CH,), jnp.int32),
            pltpu.VMEM((2 * FL,), jnp.int32),
            pltpu.VMEM((2 * FL,), jnp.int32),
            pltpu.VMEM((16,), jnp.int32),
        ],
    )
    return f(edge_index)


def _segmax_body(hp_hbm, srcl_hbm, dstl_hbm, cnt_hbm, agg_hbm,
                 sidx, dloc, rows, acc0, acc1, acc2, acc3, cnt_v, sem):
    w = _wid()
    accs = (acc0, acc1, acc2, acc3)

    zero16 = jnp.zeros((16,), jnp.float32)

    def z(i, _):
        for a in accs:
            a.at[i][pl.ds(0, 16)] = zero16
        return 0

    lax.fori_loop(0, RNG + 8, z, 0)

    pltpu.sync_copy(cnt_hbm.at[pl.ds(pl.multiple_of(w * 16, 16), 16)], cnt_v)
    n = cnt_v[pl.ds(0, 16)][0]
    nblk = (n + KB - 1) // KB

    def blk(b, _):
        off = pl.multiple_of(b * KB, 8)
        loff = pl.multiple_of(w * CAP + off, 8)
        pltpu.sync_copy(srcl_hbm.at[pl.ds(loff, KB)], sidx)
        pltpu.sync_copy(dstl_hbm.at[pl.ds(loff, KB)], dloc)

        def san(v, _):
            base = pl.multiple_of(v * 16, 16)
            pos = off + base + lax.iota(jnp.int32, 16)
            ok = pos < n
            s16 = sidx[pl.ds(base, 16)]
            d16 = dloc[pl.ds(base, 16)]
            sidx[pl.ds(base, 16)] = jnp.where(ok, s16, 0)
            dloc[pl.ds(base, 16)] = jnp.where(ok, d16, RNG)
            return 0

        lax.fori_loop(0, KB // 16, san, 0)

        pltpu.async_copy(hp_hbm.at[sidx], rows, sem).wait()

        def grp(g, _):
            base = pl.multiple_of(g * 16, 16)
            d16 = dloc[pl.ds(base, 16)]
            for j in range(16):
                d = d16[j]
                r = rows.at[base + j]
                for q in range(4):
                    a = accs[q].at[d]
                    sl = pl.ds(0, 16)
                    a[sl] = jnp.maximum(a[sl], r[pl.ds(q * 16, 16)])
            return 0

        lax.fori_loop(0, KB // 16, grp, 0)
        return 0

    lax.fori_loop(0, nblk, blk, 0)

    row0 = pl.multiple_of(w * RNG, 8)
    for q in range(4):
        pltpu.sync_copy(accs[q].at[pl.ds(0, RNG)],
                        agg_hbm.at[q, pl.ds(row0, RNG)])


def _segmax(hp, srcl, dstl, cnts):
    f = pl.kernel(
        _segmax_body,
        compiler_params=pltpu.CompilerParams(needs_layout_passes=False,
                                             use_tc_tiling_on_sc=False),
        out_type=jax.ShapeDtypeStruct((4, NPAD, 16), jnp.float32),
        mesh=_MESH,
        scratch_types=[
            pltpu.VMEM((KB,), jnp.int32),
            pltpu.VMEM((KB,), jnp.int32),
            pltpu.VMEM((KB, 64), jnp.float32),
            pltpu.VMEM((RNG + 8, 16), jnp.float32),
            pltpu.VMEM((RNG + 8, 16), jnp.float32),
            pltpu.VMEM((RNG + 8, 16), jnp.float32),
            pltpu.VMEM((RNG + 8, 16), jnp.float32),
            pltpu.VMEM((16,), jnp.int32),
            pltpu.SemaphoreType.DMA,
        ],
    )
    return f(hp, srcl, dstl, cnts)


def _segsum_body(edge_hbm, h2s_hbm, sum_hbm, deg_hbm,
                 sidx, didx, rows, ones_v, zrow, zdeg, acc_sp, deg_sp, sem):
    c = lax.axis_index("c")
    s = lax.axis_index("s")

    zero16 = jnp.zeros((16,), jnp.float32)
    one16 = jnp.ones((16,), jnp.float32)

    def zinit2(i, _):
        row = zrow.at[i]
        row[pl.ds(0, 16)] = zero16
        zdeg[pl.ds(i * 16, 16)] = zero16
        return 0

    lax.fori_loop(0, ZR, zinit2, 0)

    def oinit(i, _):
        ones_v[pl.ds(i * 16, 16)] = one16
        return 0

    lax.fori_loop(0, K3 // 16, oinit, 0)

    zn = ZR * 16

    def zacc(i, _):
        pltpu.sync_copy(zrow, acc_sp.at[pl.ds(pl.multiple_of(s * zn + i * ZR, 4), ZR)])
        return 0

    lax.fori_loop(0, 16, zacc, 0)
    pltpu.sync_copy(zdeg, deg_sp.at[pl.ds(pl.multiple_of(s * zn, 8), zn)])
    plsc.subcore_barrier()

    def blk(b, _):
        off = pl.multiple_of(s * E_PER_TILE + b * K3, 8)
        pltpu.sync_copy(edge_hbm.at[pl.ds(off, K3)], sidx)
        pltpu.sync_copy(edge_hbm.at[pl.ds(pl.multiple_of(E + off, 8), K3)],
                        didx)
        pltpu.async_copy(h2s_hbm.at[c].at[sidx], rows, sem).wait()
        pltpu.sync_copy(rows, acc_sp.at[didx], add=True)
        pltpu.sync_copy(ones_v, deg_sp.at[didx], add=True)
        return 0

    lax.fori_loop(0, E_PER_TILE // K3, blk, 0)
    plsc.subcore_barrier()

    zoff = pl.multiple_of(s * zn, 8)
    pltpu.sync_copy(acc_sp.at[pl.ds(zoff, zn)],
                    sum_hbm.at[c, pl.ds(zoff, zn)])
    pltpu.sync_copy(deg_sp.at[pl.ds(zoff, zn)],
                    deg_hbm.at[pl.ds(pl.multiple_of(c * NPD + s * zn, 8), zn)])


def _segsum(edge_index, h2s):
    f = pl.kernel(
        _segsum_body,
        compiler_params=pltpu.CompilerParams(needs_layout_passes=False,
                                             use_tc_tiling_on_sc=False),
        out_type=[
            jax.ShapeDtypeStruct((NC, NPD, 16), jnp.float32),
            jax.ShapeDtypeStruct((NC * NPD,), jnp.float32),
        ],
        mesh=_MESH,
        scratch_types=[
            pltpu.VMEM((K3,), jnp.int32),
            pltpu.VMEM((K3,), jnp.int32),
            pltpu.VMEM((K3, 16), jnp.float32),
            pltpu.VMEM((K3,), jnp.float32),
            pltpu.VMEM((ZR, 16), jnp.float32),
            pltpu.VMEM((ZR * 16,), jnp.float32),
            pltpu.VMEM_SHARED((NPD, 16), jnp.float32),
            pltpu.VMEM_SHARED((NPD,), jnp.float32),
            pltpu.SemaphoreType.DMA,
        ],
    )
    return f(edge_index, h2s)


TB = 1000


def _tc_pool_in_body(x_ref, wp_ref, bp_ref, o_ref):
    o_ref[...] = jax.nn.relu(
        jnp.dot(x_ref[...], wp_ref[...], preferred_element_type=jnp.float32)
        + bp_ref[...])


def _tc_pool_in(x, Wp, bp):
    return pl.pallas_call(
        _tc_pool_in_body,
        grid=(N // TB,),
        in_specs=[
            pl.BlockSpec((TB, 64), lambda i: (i, 0)),
            pl.BlockSpec((64, 64), lambda i: (0, 0)),
            pl.BlockSpec((1, 64), lambda i: (0, 0)),
        ],
        out_specs=pl.BlockSpec((TB, 64), lambda i: (i, 0)),
        out_shape=jax.ShapeDtypeStruct((N, 64), jnp.float32),
    )(x, Wp, bp.reshape(1, 64))


def _tc_mid_body(x_ref, agg_ref, ws_ref, wn_ref, b_ref, wp_ref, bp_ref,
                 h1_ref, hp2_ref):
    agg = jnp.concatenate([agg_ref[0], agg_ref[1], agg_ref[2], agg_ref[3]],
                          axis=-1)
    h1 = jax.nn.relu(
        jnp.dot(x_ref[...], ws_ref[...], preferred_element_type=jnp.float32)
        + jnp.dot(agg, wn_ref[...], preferred_element_type=jnp.float32)
        + b_ref[...])
    h1_ref[...] = h1
    hp2_ref[...] = jax.nn.relu(
        jnp.dot(h1, wp_ref[...], preferred_element_type=jnp.float32)
        + bp_ref[...])


def _tc_mid(x, agg1, Ws1, Wn1, b1, Wp2, bp2):
    return pl.pallas_call(
        _tc_mid_body,
        grid=(N // TB,),
        in_specs=[
            pl.BlockSpec((TB, 64), lambda i: (i, 0)),
            pl.BlockSpec((4, TB, 16), lambda i: (0, i, 0)),
            pl.BlockSpec((64, 64), lambda i: (0, 0)),
            pl.BlockSpec((64, 64), lambda i: (0, 0)),
            pl.BlockSpec((1, 64), lambda i: (0, 0)),
            pl.BlockSpec((64, 64), lambda i: (0, 0)),
            pl.BlockSpec((1, 64), lambda i: (0, 0)),
        ],
        out_specs=[
            pl.BlockSpec((TB, 64), lambda i: (i, 0)),
            pl.BlockSpec((TB, 64), lambda i: (i, 0)),
        ],
        out_shape=[
            jax.ShapeDtypeStruct((N, 64), jnp.float32),
            jax.ShapeDtypeStruct((N, 64), jnp.float32),
        ],
    )(x, agg1, Ws1, Wn1, b1.reshape(1, 64), Wp2, bp2.reshape(1, 64))


def _tc_h2_body(h1_ref, agg_ref, ws_ref, wn_ref, b_ref, h2_ref, h2s_ref):
    agg = jnp.concatenate([agg_ref[0], agg_ref[1], agg_ref[2], agg_ref[3]],
                          axis=-1)
    h2 = (jnp.dot(h1_ref[...], ws_ref[...], preferred_element_type=jnp.float32)
          + jnp.dot(agg, wn_ref[...],
                    preferred_element_type=jnp.float32)
          + b_ref[...])
    h2_ref[...] = h2
    h2s_ref[0] = h2[:, :16]
    h2s_ref[1] = h2[:, 16:]


def _tc_h2(h1, agg2, Ws2, Wn2, b2):
    return pl.pallas_call(
        _tc_h2_body,
        grid=(N // TB,),
        in_specs=[
            pl.BlockSpec((TB, 64), lambda i: (i, 0)),
            pl.BlockSpec((4, TB, 16), lambda i: (0, i, 0)),
            pl.BlockSpec((64, 32), lambda i: (0, 0)),
            pl.BlockSpec((64, 32), lambda i: (0, 0)),
            pl.BlockSpec((1, 32), lambda i: (0, 0)),
        ],
        out_specs=[
            pl.BlockSpec((TB, 32), lambda i: (i, 0)),
            pl.BlockSpec((2, TB, 16), lambda i: (0, i, 0)),
        ],
        out_shape=[
            jax.ShapeDtypeStruct((N, 32), jnp.float32),
            jax.ShapeDtypeStruct((2, N, 16), jnp.float32),
        ],
    )(h1, agg2, Ws2, Wn2, b2.reshape(1, 32))


def _tc_out_body(h2_ref, s_ref, deg_ref, ws_ref, wn_ref, b_ref, o_ref):
    ssum = jnp.concatenate([s_ref[0], s_ref[1]], axis=-1)
    deg = deg_ref[0]
    mean = ssum / jnp.maximum(deg, 1.0)
    o_ref[...] = (
        jnp.dot(h2_ref[...], ws_ref[...], preferred_element_type=jnp.float32)
        + jnp.dot(mean, wn_ref[...], preferred_element_type=jnp.float32)
        + b_ref[...])


def _tc_out(h2, ssum, deg, Ws3, Wn3, b3):
    return pl.pallas_call(
        _tc_out_body,
        grid=(N // TB,),
        in_specs=[
            pl.BlockSpec((TB, 32), lambda i: (i, 0)),
            pl.BlockSpec((2, TB, 16), lambda i: (0, i, 0)),
            pl.BlockSpec((1, TB, 1), lambda i: (0, i, 0)),
            pl.BlockSpec((32, 32), lambda i: (0, 0)),
            pl.BlockSpec((32, 32), lambda i: (0, 0)),
            pl.BlockSpec((1, 32), lambda i: (0, 0)),
        ],
        out_specs=pl.BlockSpec((TB, 32), lambda i: (i, 0)),
        out_shape=jax.ShapeDtypeStruct((N, 32), jnp.float32),
    )(h2, ssum, deg.reshape(1, -1, 1), Ws3, Wn3, b3.reshape(1, 32))


def kernel(x, edge_index, Wp1, bp1, Ws1, Wn1, b1, Wp2, bp2, Ws2, Wn2, b2,
           Ws3, Wn3, b3):
    edge_flat = edge_index.reshape(2 * E)
    srcl, dstl, cnts = _partition(edge_flat)
    hp1 = _tc_pool_in(x, Wp1, bp1)
    agg1 = _segmax(hp1, srcl, dstl, cnts)
    h1, hp2 = _tc_mid(x, agg1, Ws1, Wn1, b1, Wp2, bp2)
    agg2 = _segmax(hp2, srcl, dstl, cnts)
    h2, h2s = _tc_h2(h1, agg2, Ws2, Wn2, b2)
    ssum, degs = _segsum(edge_flat, h2s)
    out = _tc_out(h2, ssum, degs, Ws3, Wn3, b3)
    return out


# speedup vs baseline: 1.2176x; 1.2176x over previous
import functools

import jax
import jax.numpy as jnp
from jax import lax
from jax.experimental import pallas as pl
from jax.experimental.pallas import tpu as pltpu
from jax.experimental.pallas import tpu_sc as plsc

N = 50000
E = 800000
NC = 2
NS = 16
NW = NC * NS
RNG = 1568
NPAD = RNG * NW

CH = 2000
FL = 2048
CAP = E + 2 * FL

KB = 192
E_PER_TILE = E // NS
K3 = 2000
ZR = 196
NPD = NS * ZR * 16

_MESH = plsc.VectorSubcoreMesh(core_axis_name="c", subcore_axis_name="s")


def _wid():
    return lax.axis_index("s") * NC + lax.axis_index("c")


def _partition_body(edge_hbm, srcl_hbm, dstl_hbm, cnt_hbm,
                    sbuf0, dbuf0, sbuf1, dbuf1, stg_s, stg_d, cnt_v,
                    semA, semB):
    w = _wid()
    lo = w * RNG
    hi = lo + RNG

    RMASK = 2 * FL - 1
    NCHUNK = E // CH

    def issue(ci, sb, db, sem):
        off = pl.multiple_of(ci * CH, 8)
        pltpu.async_copy(edge_hbm.at[pl.ds(off, CH)], sb, sem)
        pltpu.async_copy(edge_hbm.at[pl.ds(pl.multiple_of(E + off, 8), CH)],
                         db, sem)

    def drain(sb, db, sem):
        pltpu.make_async_copy(edge_hbm.at[pl.ds(0, CH)], sb, sem).wait()
        pltpu.make_async_copy(edge_hbm.at[pl.ds(0, CH)], db, sem).wait()

    def process(sb, db, carry):
        fill, oo = carry

        def vec(vi, fill):
            base = fill
            for k in range(25):
                b = pl.multiple_of(vi * 400 + k * 16, 16)
                s16 = sb[pl.ds(b, 16)]
                d16 = db[pl.ds(b, 16)]
                m = (d16 >= lo) & (d16 < hi)
                mi = m.astype(jnp.int32)
                cs = plsc.cumsum(mi)
                dest = (base + cs - mi) & RMASK
                plsc.store_scatter(stg_s, [dest], s16, mask=m)
                plsc.store_scatter(stg_d, [dest], d16 - lo, mask=m)
                base = base + cs[15]
            return base

        fill = lax.fori_loop(0, CH // 400, vec, fill)

        def do_flush(args):
            fill, oo = args
            fp = pl.multiple_of((oo & RMASK), FL)
            foff = pl.multiple_of(w * CAP + oo, 8)
            pltpu.sync_copy(stg_s.at[pl.ds(fp, FL)],
                            srcl_hbm.at[pl.ds(foff, FL)])
            pltpu.sync_copy(stg_d.at[pl.ds(fp, FL)],
                            dstl_hbm.at[pl.ds(foff, FL)])
            return fill, oo + FL

        return lax.cond(fill - oo >= FL, do_flush, lambda a: a, (fill, oo))

    issue(0, sbuf0, dbuf0, semA)
    issue(1, sbuf1, dbuf1, semB)

    def pair(pi, carry):
        drain(sbuf0, dbuf0, semA)
        carry = process(sbuf0, dbuf0, carry)
        nxtA = jnp.minimum(2 * pi + 2, NCHUNK - 1)
        issue(nxtA, sbuf0, dbuf0, semA)
        drain(sbuf1, dbuf1, semB)
        carry = process(sbuf1, dbuf1, carry)
        nxtB = jnp.minimum(2 * pi + 3, NCHUNK - 1)
        issue(nxtB, sbuf1, dbuf1, semB)
        return carry

    fill, oo = lax.fori_loop(0, NCHUNK // 2, pair, (0, 0))
    drain(sbuf0, dbuf0, semA)
    drain(sbuf1, dbuf1, semB)
    fp = pl.multiple_of((oo & RMASK), FL)
    foff = pl.multiple_of(w * CAP + oo, 8)
    pltpu.sync_copy(stg_s.at[pl.ds(fp, FL)], srcl_hbm.at[pl.ds(foff, FL)])
    pltpu.sync_copy(stg_d.at[pl.ds(fp, FL)], dstl_hbm.at[pl.ds(foff, FL)])
    cnt_v[pl.ds(0, 16)] = jnp.full((16,), fill, jnp.int32)
    pltpu.sync_copy(cnt_v, cnt_hbm.at[pl.ds(pl.multiple_of(w * 16, 16), 16)])


def _partition(edge_index):
    f = pl.kernel(
        _partition_body,
        compiler_params=pltpu.CompilerParams(needs_layout_passes=False,
                                             use_tc_tiling_on_sc=False),
        out_type=[
            jax.ShapeDtypeStruct((NW * CAP,), jnp.int32),
            jax.ShapeDtypeStruct((NW * CAP,), jnp.int32),
            jax.ShapeDtypeStruct((NW * 16,), jnp.int32),
        ],
        mesh=_MESH,
        scratch_types=[
            pltpu.VMEM((CH,), jnp.int32),
            pltpu.VMEM((CH,), jnp.int32),
            pltpu.VMEM((CH,), jnp.int32),
            pltpu.VMEM((CH,), jnp.int32),
            pltpu.VMEM((2 * FL,), jnp.int32),
            pltpu.VMEM((2 * FL,), jnp.int32),
            pltpu.VMEM((16,), jnp.int32),
            pltpu.SemaphoreType.DMA,
            pltpu.SemaphoreType.DMA,
        ],
    )
    return f(edge_index)


def _segmax_body(hp_hbm, srcl_hbm, dstl_hbm, cnt_hbm, agg_hbm,
                 sidxA, dlocA, sidxB, dlocB, dprA, dprB, rowsA, rowsB,
                 acc0, acc1, acc2, acc3, cnt_v,
                 siA, siB, sgA, sgB):
    w = _wid()
    accs = (acc0, acc1, acc2, acc3)

    zero16 = jnp.zeros((16,), jnp.float32)

    def z(i, _):
        for a in accs:
            a.at[i][pl.ds(0, 16)] = zero16
        return 0

    lax.fori_loop(0, RNG + 8, z, 0)

    pltpu.sync_copy(cnt_hbm.at[pl.ds(pl.multiple_of(w * 16, 16), 16)], cnt_v)
    n = cnt_v[pl.ds(0, 16)][0]
    nblk = (n + KB - 1) // KB
    nbp = (nblk + 1) // 2

    def issue_idx(b, si, dl, sem):
        off = pl.multiple_of(b * KB, 8)
        loff = pl.multiple_of(w * CAP + off, 8)
        pltpu.async_copy(srcl_hbm.at[pl.ds(loff, KB)], si, sem)
        pltpu.async_copy(dstl_hbm.at[pl.ds(loff, KB)], dl, sem)

    def wait_idx(si, dl, sem):
        pltpu.make_async_copy(srcl_hbm.at[pl.ds(0, KB)], si, sem).wait()
        pltpu.make_async_copy(dstl_hbm.at[pl.ds(0, KB)], dl, sem).wait()

    def sanitize_and_gather(b, si, dl, dpr, rows, sem):
        off = b * KB

        def san(v, _):
            base = pl.multiple_of(v * 16, 16)
            pos = off + base + lax.iota(jnp.int32, 16)
            ok = pos < n
            s16 = si[pl.ds(base, 16)]
            d16 = dl[pl.ds(base, 16)]
            si[pl.ds(base, 16)] = jnp.where(ok, s16, 0)
            dpr[pl.ds(base, 16)] = jnp.where(ok, d16, RNG)
            return 0

        lax.fori_loop(0, KB // 16, san, 0)
        pltpu.async_copy(hp_hbm.at[si], rows, sem)

    def wait_rows(rows, sem):
        pltpu.make_async_copy(hp_hbm.at[pl.ds(0, KB)], rows, sem).wait()

    def process(dl, rows):
        def grp(g, _):
            base = pl.multiple_of(g * 16, 16)
            d16 = dl[pl.ds(base, 16)]
            for j in range(16):
                d = d16[j]
                r = rows.at[base + j]
                for q in range(4):
                    a = accs[q].at[d]
                    a[pl.ds(0, 16)] = jnp.maximum(a[pl.ds(0, 16)],
                                                  r[pl.ds(q * 16, 16)])
            return 0

        lax.fori_loop(0, KB // 16, grp, 0)

    issue_idx(0, sidxA, dlocA, siA)
    issue_idx(1, sidxB, dlocB, siB)
    wait_idx(sidxA, dlocA, siA)
    sanitize_and_gather(0, sidxA, dlocA, dprA, rowsA, sgA)

    def pair(pi, _):
        b = 2 * pi
        wait_idx(sidxB, dlocB, siB)
        sanitize_and_gather(b + 1, sidxB, dlocB, dprB, rowsB, sgB)
        wait_rows(rowsA, sgA)
        issue_idx(b + 2, sidxA, dlocA, siA)
        process(dprA, rowsA)
        wait_idx(sidxA, dlocA, siA)
        sanitize_and_gather(b + 2, sidxA, dlocA, dprA, rowsA, sgA)
        wait_rows(rowsB, sgB)
        issue_idx(b + 3, sidxB, dlocB, siB)
        process(dprB, rowsB)
        return 0

    lax.fori_loop(0, nbp, pair, 0)
    wait_rows(rowsA, sgA)
    wait_idx(sidxB, dlocB, siB)

    row0 = pl.multiple_of(w * RNG, 8)
    for q in range(4):
        pltpu.sync_copy(accs[q].at[pl.ds(0, RNG)],
                        agg_hbm.at[q, pl.ds(row0, RNG)])


def _segmax(hp, srcl, dstl, cnts):
    f = pl.kernel(
        _segmax_body,
        compiler_params=pltpu.CompilerParams(needs_layout_passes=False,
                                             use_tc_tiling_on_sc=False),
        out_type=jax.ShapeDtypeStruct((4, NPAD, 16), jnp.float32),
        mesh=_MESH,
        scratch_types=[
            pltpu.VMEM((KB,), jnp.int32),
            pltpu.VMEM((KB,), jnp.int32),
            pltpu.VMEM((KB,), jnp.int32),
            pltpu.VMEM((KB,), jnp.int32),
            pltpu.VMEM((KB,), jnp.int32),
            pltpu.VMEM((KB,), jnp.int32),
            pltpu.VMEM((KB, 64), jnp.float32),
            pltpu.VMEM((KB, 64), jnp.float32),
            pltpu.VMEM((RNG + 8, 16), jnp.float32),
            pltpu.VMEM((RNG + 8, 16), jnp.float32),
            pltpu.VMEM((RNG + 8, 16), jnp.float32),
            pltpu.VMEM((RNG + 8, 16), jnp.float32),
            pltpu.VMEM((16,), jnp.int32),
            pltpu.SemaphoreType.DMA,
            pltpu.SemaphoreType.DMA,
            pltpu.SemaphoreType.DMA,
            pltpu.SemaphoreType.DMA,
        ],
    )
    return f(hp, srcl, dstl, cnts)


def _segsum_body(edge_hbm, h2s_hbm, sum_hbm, deg_hbm,
                 sidx, didx, rows, ones_v, zrow, zdeg, acc_sp, deg_sp, sem):
    c = lax.axis_index("c")
    s = lax.axis_index("s")

    zero16 = jnp.zeros((16,), jnp.float32)
    one16 = jnp.ones((16,), jnp.float32)

    def zinit2(i, _):
        row = zrow.at[i]
        row[pl.ds(0, 16)] = zero16
        zdeg[pl.ds(i * 16, 16)] = zero16
        return 0

    lax.fori_loop(0, ZR, zinit2, 0)

    def oinit(i, _):
        ones_v[pl.ds(i * 16, 16)] = one16
        return 0

    lax.fori_loop(0, K3 // 16, oinit, 0)

    zn = ZR * 16

    def zacc(i, _):
        pltpu.sync_copy(zrow, acc_sp.at[pl.ds(pl.multiple_of(s * zn + i * ZR, 4), ZR)])
        return 0

    lax.fori_loop(0, 16, zacc, 0)
    pltpu.sync_copy(zdeg, deg_sp.at[pl.ds(pl.multiple_of(s * zn, 8), zn)])
    plsc.subcore_barrier()

    def blk(b, _):
        off = pl.multiple_of(s * E_PER_TILE + b * K3, 8)
        pltpu.sync_copy(edge_hbm.at[pl.ds(off, K3)], sidx)
        pltpu.sync_copy(edge_hbm.at[pl.ds(pl.multiple_of(E + off, 8), K3)],
                        didx)
        pltpu.async_copy(h2s_hbm.at[c].at[sidx], rows, sem).wait()
        pltpu.sync_copy(rows, acc_sp.at[didx], add=True)
        pltpu.sync_copy(ones_v, deg_sp.at[didx], add=True)
        return 0

    lax.fori_loop(0, E_PER_TILE // K3, blk, 0)
    plsc.subcore_barrier()

    zoff = pl.multiple_of(s * zn, 8)
    pltpu.sync_copy(acc_sp.at[pl.ds(zoff, zn)],
                    sum_hbm.at[c, pl.ds(zoff, zn)])
    pltpu.sync_copy(deg_sp.at[pl.ds(zoff, zn)],
                    deg_hbm.at[pl.ds(pl.multiple_of(c * NPD + s * zn, 8), zn)])


def _segsum(edge_index, h2s):
    f = pl.kernel(
        _segsum_body,
        compiler_params=pltpu.CompilerParams(needs_layout_passes=False,
                                             use_tc_tiling_on_sc=False),
        out_type=[
            jax.ShapeDtypeStruct((NC, NPD, 16), jnp.float32),
            jax.ShapeDtypeStruct((NC * NPD,), jnp.float32),
        ],
        mesh=_MESH,
        scratch_types=[
            pltpu.VMEM((K3,), jnp.int32),
            pltpu.VMEM((K3,), jnp.int32),
            pltpu.VMEM((K3, 16), jnp.float32),
            pltpu.VMEM((K3,), jnp.float32),
            pltpu.VMEM((ZR, 16), jnp.float32),
            pltpu.VMEM((ZR * 16,), jnp.float32),
            pltpu.VMEM_SHARED((NPD, 16), jnp.float32),
            pltpu.VMEM_SHARED((NPD,), jnp.float32),
            pltpu.SemaphoreType.DMA,
        ],
    )
    return f(edge_index, h2s)


TB = 1000


def _tc_pool_in_body(x_ref, wp_ref, bp_ref, o_ref):
    o_ref[...] = jax.nn.relu(
        jnp.dot(x_ref[...], wp_ref[...], preferred_element_type=jnp.float32)
        + bp_ref[...])


def _tc_pool_in(x, Wp, bp):
    return pl.pallas_call(
        _tc_pool_in_body,
        grid=(N // TB,),
        in_specs=[
            pl.BlockSpec((TB, 64), lambda i: (i, 0)),
            pl.BlockSpec((64, 64), lambda i: (0, 0)),
            pl.BlockSpec((1, 64), lambda i: (0, 0)),
        ],
        out_specs=pl.BlockSpec((TB, 64), lambda i: (i, 0)),
        out_shape=jax.ShapeDtypeStruct((N, 64), jnp.float32),
    )(x, Wp, bp.reshape(1, 64))


def _tc_mid_body(x_ref, agg_ref, ws_ref, wn_ref, b_ref, wp_ref, bp_ref,
                 h1_ref, hp2_ref):
    agg = jnp.concatenate([agg_ref[0], agg_ref[1], agg_ref[2], agg_ref[3]],
                          axis=-1)
    h1 = jax.nn.relu(
        jnp.dot(x_ref[...], ws_ref[...], preferred_element_type=jnp.float32)
        + jnp.dot(agg, wn_ref[...], preferred_element_type=jnp.float32)
        + b_ref[...])
    h1_ref[...] = h1
    hp2_ref[...] = jax.nn.relu(
        jnp.dot(h1, wp_ref[...], preferred_element_type=jnp.float32)
        + bp_ref[...])


def _tc_mid(x, agg1, Ws1, Wn1, b1, Wp2, bp2):
    return pl.pallas_call(
        _tc_mid_body,
        grid=(N // TB,),
        in_specs=[
            pl.BlockSpec((TB, 64), lambda i: (i, 0)),
            pl.BlockSpec((4, TB, 16), lambda i: (0, i, 0)),
            pl.BlockSpec((64, 64), lambda i: (0, 0)),
            pl.BlockSpec((64, 64), lambda i: (0, 0)),
            pl.BlockSpec((1, 64), lambda i: (0, 0)),
            pl.BlockSpec((64, 64), lambda i: (0, 0)),
            pl.BlockSpec((1, 64), lambda i: (0, 0)),
        ],
        out_specs=[
            pl.BlockSpec((TB, 64), lambda i: (i, 0)),
            pl.BlockSpec((TB, 64), lambda i: (i, 0)),
        ],
        out_shape=[
            jax.ShapeDtypeStruct((N, 64), jnp.float32),
            jax.ShapeDtypeStruct((N, 64), jnp.float32),
        ],
    )(x, agg1, Ws1, Wn1, b1.reshape(1, 64), Wp2, bp2.reshape(1, 64))


def _tc_h2_body(h1_ref, agg_ref, ws_ref, wn_ref, b_ref, h2_ref, h2s_ref):
    agg = jnp.concatenate([agg_ref[0], agg_ref[1], agg_ref[2], agg_ref[3]],
                          axis=-1)
    h2 = (jnp.dot(h1_ref[...], ws_ref[...], preferred_element_type=jnp.float32)
          + jnp.dot(agg, wn_ref[...],
                    preferred_element_type=jnp.float32)
          + b_ref[...])
    h2_ref[...] = h2
    h2s_ref[0] = h2[:, :16]
    h2s_ref[1] = h2[:, 16:]


def _tc_h2(h1, agg2, Ws2, Wn2, b2):
    return pl.pallas_call(
        _tc_h2_body,
        grid=(N // TB,),
        in_specs=[
            pl.BlockSpec((TB, 64), lambda i: (i, 0)),
            pl.BlockSpec((4, TB, 16), lambda i: (0, i, 0)),
            pl.BlockSpec((64, 32), lambda i: (0, 0)),
            pl.BlockSpec((64, 32), lambda i: (0, 0)),
            pl.BlockSpec((1, 32), lambda i: (0, 0)),
        ],
        out_specs=[
            pl.BlockSpec((TB, 32), lambda i: (i, 0)),
            pl.BlockSpec((2, TB, 16), lambda i: (0, i, 0)),
        ],
        out_shape=[
            jax.ShapeDtypeStruct((N, 32), jnp.float32),
            jax.ShapeDtypeStruct((2, N, 16), jnp.float32),
        ],
    )(h1, agg2, Ws2, Wn2, b2.reshape(1, 32))


def _tc_out_body(h2_ref, s_ref, deg_ref, ws_ref, wn_ref, b_ref, o_ref):
    ssum = jnp.concatenate([s_ref[0], s_ref[1]], axis=-1)
    deg = deg_ref[0]
    mean = ssum / jnp.maximum(deg, 1.0)
    o_ref[...] = (
        jnp.dot(h2_ref[...], ws_ref[...], preferred_element_type=jnp.float32)
        + jnp.dot(mean, wn_ref[...], preferred_element_type=jnp.float32)
        + b_ref[...])


def _tc_out(h2, ssum, deg, Ws3, Wn3, b3):
    return pl.pallas_call(
        _tc_out_body,
        grid=(N // TB,),
        in_specs=[
            pl.BlockSpec((TB, 32), lambda i: (i, 0)),
            pl.BlockSpec((2, TB, 16), lambda i: (0, i, 0)),
            pl.BlockSpec((1, TB, 1), lambda i: (0, i, 0)),
            pl.BlockSpec((32, 32), lambda i: (0, 0)),
            pl.BlockSpec((32, 32), lambda i: (0, 0)),
            pl.BlockSpec((1, 32), lambda i: (0, 0)),
        ],
        out_specs=pl.BlockSpec((TB, 32), lambda i: (i, 0)),
        out_shape=jax.ShapeDtypeStruct((N, 32), jnp.float32),
    )(h2, ssum, deg.reshape(1, -1, 1), Ws3, Wn3, b3.reshape(1, 32))


def kernel(x, edge_index, Wp1, bp1, Ws1, Wn1, b1, Wp2, bp2, Ws2, Wn2, b2,
           Ws3, Wn3, b3):
    edge_flat = edge_index.reshape(2 * E)
    srcl, dstl, cnts = _partition(edge_flat)
    hp1 = _tc_pool_in(x, Wp1, bp1)
    agg1 = _segmax(hp1, srcl, dstl, cnts)
    h1, hp2 = _tc_mid(x, agg1, Ws1, Wn1, b1, Wp2, bp2)
    agg2 = _segmax(hp2, srcl, dstl, cnts)
    h2, h2s = _tc_h2(h1, agg2, Ws2, Wn2, b2)
    ssum, degs = _segsum(edge_flat, h2s)
    out = _tc_out(h2, ssum, degs, Ws3, Wn3, b3)
    return out


# speedup vs baseline: 1.2292x; 1.0095x over previous
import functools

import jax
import jax.numpy as jnp
from jax import lax
from jax.experimental import pallas as pl
from jax.experimental.pallas import tpu as pltpu
from jax.experimental.pallas import tpu_sc as plsc

N = 50000
E = 800000
NC = 2
NS = 16
NW = NC * NS
RNG = 1568
NPAD = RNG * NW

CH = 2000
FL = 2048
CAP = E + 2 * FL

KB = 192
E_PER_TILE = E // NS
K3 = 2000
ZR = 196
NPD = NS * ZR * 16

_MESH = plsc.VectorSubcoreMesh(core_axis_name="c", subcore_axis_name="s")


def _wid():
    return lax.axis_index("s") * NC + lax.axis_index("c")


def _partition_body(edge_hbm, srcl_hbm, cnt_hbm,
                    sbuf0, dbuf0, sbuf1, dbuf1, stg_s, cnt_v,
                    semA, semB):
    w = _wid()
    lo = w * RNG
    hi = lo + RNG

    RMASK = 2 * FL - 1
    NCHUNK = E // CH

    def issue(ci, sb, db, sem):
        off = pl.multiple_of(ci * CH, 8)
        pltpu.async_copy(edge_hbm.at[pl.ds(off, CH)], sb, sem)
        pltpu.async_copy(edge_hbm.at[pl.ds(pl.multiple_of(E + off, 8), CH)],
                         db, sem)

    def drain(sb, db, sem):
        pltpu.make_async_copy(edge_hbm.at[pl.ds(0, CH)], sb, sem).wait()
        pltpu.make_async_copy(edge_hbm.at[pl.ds(0, CH)], db, sem).wait()

    def process(sb, db, carry):
        fill, oo = carry

        def vec(vi, fill):
            base = fill
            for k in range(25):
                b = pl.multiple_of(vi * 400 + k * 16, 16)
                s16 = sb[pl.ds(b, 16)]
                d16 = db[pl.ds(b, 16)]
                m = (d16 >= lo) & (d16 < hi)
                mi = m.astype(jnp.int32)
                cs = plsc.cumsum(mi)
                dest = (base + cs - mi) & RMASK
                packed = ((d16 - lo) << 16) | s16
                plsc.store_scatter(stg_s, [dest], packed, mask=m)
                base = base + cs[15]
            return base

        fill = lax.fori_loop(0, CH // 400, vec, fill)

        def do_flush(args):
            fill, oo = args
            fp = pl.multiple_of((oo & RMASK), FL)
            foff = pl.multiple_of(w * CAP + oo, 8)
            pltpu.sync_copy(stg_s.at[pl.ds(fp, FL)],
                            srcl_hbm.at[pl.ds(foff, FL)])
            return fill, oo + FL

        return lax.cond(fill - oo >= FL, do_flush, lambda a: a, (fill, oo))

    issue(0, sbuf0, dbuf0, semA)
    issue(1, sbuf1, dbuf1, semB)

    def pair(pi, carry):
        drain(sbuf0, dbuf0, semA)
        carry = process(sbuf0, dbuf0, carry)
        nxtA = jnp.minimum(2 * pi + 2, NCHUNK - 1)
        issue(nxtA, sbuf0, dbuf0, semA)
        drain(sbuf1, dbuf1, semB)
        carry = process(sbuf1, dbuf1, carry)
        nxtB = jnp.minimum(2 * pi + 3, NCHUNK - 1)
        issue(nxtB, sbuf1, dbuf1, semB)
        return carry

    fill, oo = lax.fori_loop(0, NCHUNK // 2, pair, (0, 0))
    drain(sbuf0, dbuf0, semA)
    drain(sbuf1, dbuf1, semB)
    fp = pl.multiple_of((oo & RMASK), FL)
    foff = pl.multiple_of(w * CAP + oo, 8)
    pltpu.sync_copy(stg_s.at[pl.ds(fp, FL)], srcl_hbm.at[pl.ds(foff, FL)])
    cnt_v[pl.ds(0, 16)] = jnp.full((16,), fill, jnp.int32)
    pltpu.sync_copy(cnt_v, cnt_hbm.at[pl.ds(pl.multiple_of(w * 16, 16), 16)])


def _partition(edge_index):
    f = pl.kernel(
        _partition_body,
        compiler_params=pltpu.CompilerParams(needs_layout_passes=False,
                                             use_tc_tiling_on_sc=False),
        out_type=[
            jax.ShapeDtypeStruct((NW * CAP,), jnp.int32),
            jax.ShapeDtypeStruct((NW * 16,), jnp.int32),
        ],
        mesh=_MESH,
        scratch_types=[
            pltpu.VMEM((CH,), jnp.int32),
            pltpu.VMEM((CH,), jnp.int32),
            pltpu.VMEM((CH,), jnp.int32),
            pltpu.VMEM((CH,), jnp.int32),
            pltpu.VMEM((2 * FL,), jnp.int32),
            pltpu.VMEM((16,), jnp.int32),
            pltpu.SemaphoreType.DMA,
            pltpu.SemaphoreType.DMA,
        ],
    )
    return f(edge_index)


def _segmax_body(hp_hbm, srcl_hbm, cnt_hbm, agg_hbm,
                 sidxA, sidxB, dprA, dprB, rowsA, rowsB,
                 acc0, acc1, acc2, acc3, cnt_v,
                 siA, siB, sgA, sgB):
    w = _wid()
    accs = (acc0, acc1, acc2, acc3)

    zero16 = jnp.zeros((16,), jnp.float32)

    def z(i, _):
        for a in accs:
            a.at[i][pl.ds(0, 16)] = zero16
        return 0

    lax.fori_loop(0, RNG + 8, z, 0)

    pltpu.sync_copy(cnt_hbm.at[pl.ds(pl.multiple_of(w * 16, 16), 16)], cnt_v)
    n = cnt_v[pl.ds(0, 16)][0]
    nblk = (n + KB - 1) // KB
    nbp = (nblk + 1) // 2

    def issue_idx(b, si, sem):
        off = pl.multiple_of(b * KB, 8)
        loff = pl.multiple_of(w * CAP + off, 8)
        pltpu.async_copy(srcl_hbm.at[pl.ds(loff, KB)], si, sem)

    def wait_idx(si, sem):
        pltpu.make_async_copy(srcl_hbm.at[pl.ds(0, KB)], si, sem).wait()

    def sanitize_and_gather(b, si, dpr, rows, sem):
        off = b * KB

        def san(v, _):
            base = pl.multiple_of(v * 16, 16)
            pos = off + base + lax.iota(jnp.int32, 16)
            ok = pos < n
            v16 = si[pl.ds(base, 16)]
            si[pl.ds(base, 16)] = jnp.where(ok, v16 & 0xFFFF, 0)
            dpr[pl.ds(base, 16)] = jnp.where(ok, v16 >> 16, RNG)
            return 0

        lax.fori_loop(0, KB // 16, san, 0)
        pltpu.async_copy(hp_hbm.at[si], rows, sem)

    def wait_rows(rows, sem):
        pltpu.make_async_copy(hp_hbm.at[pl.ds(0, KB)], rows, sem).wait()

    def process(dl, rows):
        def grp(g, _):
            base = pl.multiple_of(g * 16, 16)
            d16 = dl[pl.ds(base, 16)]
            for j in range(16):
                d = d16[j]
                r = rows.at[base + j]
                for q in range(4):
                    a = accs[q].at[d]
                    a[pl.ds(0, 16)] = jnp.maximum(a[pl.ds(0, 16)],
                                                  r[pl.ds(q * 16, 16)])
            return 0

        lax.fori_loop(0, KB // 16, grp, 0)

    issue_idx(0, sidxA, siA)
    issue_idx(1, sidxB, siB)
    wait_idx(sidxA, siA)
    sanitize_and_gather(0, sidxA, dprA, rowsA, sgA)

    def pair(pi, _):
        b = 2 * pi
        wait_idx(sidxB, siB)
        sanitize_and_gather(b + 1, sidxB, dprB, rowsB, sgB)
        wait_rows(rowsA, sgA)
        issue_idx(b + 2, sidxA, siA)
        process(dprA, rowsA)
        wait_idx(sidxA, siA)
        sanitize_and_gather(b + 2, sidxA, dprA, rowsA, sgA)
        wait_rows(rowsB, sgB)
        issue_idx(b + 3, sidxB, siB)
        process(dprB, rowsB)
        return 0

    lax.fori_loop(0, nbp, pair, 0)
    wait_rows(rowsA, sgA)
    wait_idx(sidxB, siB)

    row0 = pl.multiple_of(w * RNG, 8)
    for q in range(4):
        pltpu.sync_copy(accs[q].at[pl.ds(0, RNG)],
                        agg_hbm.at[q, pl.ds(row0, RNG)])


def _segmax(hp, srcl, cnts):
    f = pl.kernel(
        _segmax_body,
        compiler_params=pltpu.CompilerParams(needs_layout_passes=False,
                                             use_tc_tiling_on_sc=False),
        out_type=jax.ShapeDtypeStruct((4, NPAD, 16), jnp.float32),
        mesh=_MESH,
        scratch_types=[
            pltpu.VMEM((KB,), jnp.int32),
            pltpu.VMEM((KB,), jnp.int32),
            pltpu.VMEM((KB,), jnp.int32),
            pltpu.VMEM((KB,), jnp.int32),
            pltpu.VMEM((KB, 64), jnp.float32),
            pltpu.VMEM((KB, 64), jnp.float32),
            pltpu.VMEM((RNG + 8, 16), jnp.float32),
            pltpu.VMEM((RNG + 8, 16), jnp.float32),
            pltpu.VMEM((RNG + 8, 16), jnp.float32),
            pltpu.VMEM((RNG + 8, 16), jnp.float32),
            pltpu.VMEM((16,), jnp.int32),
            pltpu.SemaphoreType.DMA,
            pltpu.SemaphoreType.DMA,
            pltpu.SemaphoreType.DMA,
            pltpu.SemaphoreType.DMA,
        ],
    )
    return f(hp, srcl, cnts)


def _segsum_body(edge_hbm, h2s_hbm, sum_hbm, deg_hbm,
                 sidx, didx, rows, ones_v, zrow, zdeg, acc_sp, deg_sp, sem):
    c = lax.axis_index("c")
    s = lax.axis_index("s")

    zero16 = jnp.zeros((16,), jnp.float32)
    one16 = jnp.ones((16,), jnp.float32)

    def zinit2(i, _):
        row = zrow.at[i]
        row[pl.ds(0, 16)] = zero16
        zdeg[pl.ds(i * 16, 16)] = zero16
        return 0

    lax.fori_loop(0, ZR, zinit2, 0)

    def oinit(i, _):
        ones_v[pl.ds(i * 16, 16)] = one16
        return 0

    lax.fori_loop(0, K3 // 16, oinit, 0)

    zn = ZR * 16

    def zacc(i, _):
        pltpu.sync_copy(zrow, acc_sp.at[pl.ds(pl.multiple_of(s * zn + i * ZR, 4), ZR)])
        return 0

    lax.fori_loop(0, 16, zacc, 0)
    pltpu.sync_copy(zdeg, deg_sp.at[pl.ds(pl.multiple_of(s * zn, 8), zn)])
    plsc.subcore_barrier()

    def blk(b, _):
        off = pl.multiple_of(s * E_PER_TILE + b * K3, 8)
        pltpu.sync_copy(edge_hbm.at[pl.ds(off, K3)], sidx)
        pltpu.sync_copy(edge_hbm.at[pl.ds(pl.multiple_of(E + off, 8), K3)],
                        didx)
        pltpu.async_copy(h2s_hbm.at[c].at[sidx], rows, sem).wait()
        pltpu.sync_copy(rows, acc_sp.at[didx], add=True)
        pltpu.sync_copy(ones_v, deg_sp.at[didx], add=True)
        return 0

    lax.fori_loop(0, E_PER_TILE // K3, blk, 0)
    plsc.subcore_barrier()

    zoff = pl.multiple_of(s * zn, 8)
    pltpu.sync_copy(acc_sp.at[pl.ds(zoff, zn)],
                    sum_hbm.at[c, pl.ds(zoff, zn)])
    pltpu.sync_copy(deg_sp.at[pl.ds(zoff, zn)],
                    deg_hbm.at[pl.ds(pl.multiple_of(c * NPD + s * zn, 8), zn)])


def _segsum(edge_index, h2s):
    f = pl.kernel(
        _segsum_body,
        compiler_params=pltpu.CompilerParams(needs_layout_passes=False,
                                             use_tc_tiling_on_sc=False),
        out_type=[
            jax.ShapeDtypeStruct((NC, NPD, 16), jnp.float32),
            jax.ShapeDtypeStruct((NC * NPD,), jnp.float32),
        ],
        mesh=_MESH,
        scratch_types=[
            pltpu.VMEM((K3,), jnp.int32),
            pltpu.VMEM((K3,), jnp.int32),
            pltpu.VMEM((K3, 16), jnp.float32),
            pltpu.VMEM((K3,), jnp.float32),
            pltpu.VMEM((ZR, 16), jnp.float32),
            pltpu.VMEM((ZR * 16,), jnp.float32),
            pltpu.VMEM_SHARED((NPD, 16), jnp.float32),
            pltpu.VMEM_SHARED((NPD,), jnp.float32),
            pltpu.SemaphoreType.DMA,
        ],
    )
    return f(edge_index, h2s)


TB = 1000


def _tc_pool_in_body(x_ref, wp_ref, bp_ref, o_ref):
    o_ref[...] = jax.nn.relu(
        jnp.dot(x_ref[...], wp_ref[...], preferred_element_type=jnp.float32)
        + bp_ref[...])


def _tc_pool_in(x, Wp, bp):
    return pl.pallas_call(
        _tc_pool_in_body,
        grid=(N // TB,),
        in_specs=[
            pl.BlockSpec((TB, 64), lambda i: (i, 0)),
            pl.BlockSpec((64, 64), lambda i: (0, 0)),
            pl.BlockSpec((1, 64), lambda i: (0, 0)),
        ],
        out_specs=pl.BlockSpec((TB, 64), lambda i: (i, 0)),
        out_shape=jax.ShapeDtypeStruct((N, 64), jnp.float32),
    )(x, Wp, bp.reshape(1, 64))


def _tc_mid_body(x_ref, agg_ref, ws_ref, wn_ref, b_ref, wp_ref, bp_ref,
                 h1_ref, hp2_ref):
    agg = jnp.concatenate([agg_ref[0], agg_ref[1], agg_ref[2], agg_ref[3]],
                          axis=-1)
    h1 = jax.nn.relu(
        jnp.dot(x_ref[...], ws_ref[...], preferred_element_type=jnp.float32)
        + jnp.dot(agg, wn_ref[...], preferred_element_type=jnp.float32)
        + b_ref[...])
    h1_ref[...] = h1
    hp2_ref[...] = jax.nn.relu(
        jnp.dot(h1, wp_ref[...], preferred_element_type=jnp.float32)
        + bp_ref[...])


def _tc_mid(x, agg1, Ws1, Wn1, b1, Wp2, bp2):
    return pl.pallas_call(
        _tc_mid_body,
        grid=(N // TB,),
        in_specs=[
            pl.BlockSpec((TB, 64), lambda i: (i, 0)),
            pl.BlockSpec((4, TB, 16), lambda i: (0, i, 0)),
            pl.BlockSpec((64, 64), lambda i: (0, 0)),
            pl.BlockSpec((64, 64), lambda i: (0, 0)),
            pl.BlockSpec((1, 64), lambda i: (0, 0)),
            pl.BlockSpec((64, 64), lambda i: (0, 0)),
            pl.BlockSpec((1, 64), lambda i: (0, 0)),
        ],
        out_specs=[
            pl.BlockSpec((TB, 64), lambda i: (i, 0)),
            pl.BlockSpec((TB, 64), lambda i: (i, 0)),
        ],
        out_shape=[
            jax.ShapeDtypeStruct((N, 64), jnp.float32),
            jax.ShapeDtypeStruct((N, 64), jnp.float32),
        ],
    )(x, agg1, Ws1, Wn1, b1.reshape(1, 64), Wp2, bp2.reshape(1, 64))


def _tc_h2_body(h1_ref, agg_ref, ws_ref, wn_ref, b_ref, h2_ref, h2s_ref):
    agg = jnp.concatenate([agg_ref[0], agg_ref[1], agg_ref[2], agg_ref[3]],
                          axis=-1)
    h2 = (jnp.dot(h1_ref[...], ws_ref[...], preferred_element_type=jnp.float32)
          + jnp.dot(agg, wn_ref[...],
                    preferred_element_type=jnp.float32)
          + b_ref[...])
    h2_ref[...] = h2
    h2s_ref[0] = h2[:, :16]
    h2s_ref[1] = h2[:, 16:]


def _tc_h2(h1, agg2, Ws2, Wn2, b2):
    return pl.pallas_call(
        _tc_h2_body,
        grid=(N // TB,),
        in_specs=[
            pl.BlockSpec((TB, 64), lambda i: (i, 0)),
            pl.BlockSpec((4, TB, 16), lambda i: (0, i, 0)),
            pl.BlockSpec((64, 32), lambda i: (0, 0)),
            pl.BlockSpec((64, 32), lambda i: (0, 0)),
            pl.BlockSpec((1, 32), lambda i: (0, 0)),
        ],
        out_specs=[
            pl.BlockSpec((TB, 32), lambda i: (i, 0)),
            pl.BlockSpec((2, TB, 16), lambda i: (0, i, 0)),
        ],
        out_shape=[
            jax.ShapeDtypeStruct((N, 32), jnp.float32),
            jax.ShapeDtypeStruct((2, N, 16), jnp.float32),
        ],
    )(h1, agg2, Ws2, Wn2, b2.reshape(1, 32))


def _tc_out_body(h2_ref, s_ref, deg_ref, ws_ref, wn_ref, b_ref, o_ref):
    ssum = jnp.concatenate([s_ref[0], s_ref[1]], axis=-1)
    deg = deg_ref[0]
    mean = ssum / jnp.maximum(deg, 1.0)
    o_ref[...] = (
        jnp.dot(h2_ref[...], ws_ref[...], preferred_element_type=jnp.float32)
        + jnp.dot(mean, wn_ref[...], preferred_element_type=jnp.float32)
        + b_ref[...])


def _tc_out(h2, ssum, deg, Ws3, Wn3, b3):
    return pl.pallas_call(
        _tc_out_body,
        grid=(N // TB,),
        in_specs=[
            pl.BlockSpec((TB, 32), lambda i: (i, 0)),
            pl.BlockSpec((2, TB, 16), lambda i: (0, i, 0)),
            pl.BlockSpec((1, TB, 1), lambda i: (0, i, 0)),
            pl.BlockSpec((32, 32), lambda i: (0, 0)),
            pl.BlockSpec((32, 32), lambda i: (0, 0)),
            pl.BlockSpec((1, 32), lambda i: (0, 0)),
        ],
        out_specs=pl.BlockSpec((TB, 32), lambda i: (i, 0)),
        out_shape=jax.ShapeDtypeStruct((N, 32), jnp.float32),
    )(h2, ssum, deg.reshape(1, -1, 1), Ws3, Wn3, b3.reshape(1, 32))


def kernel(x, edge_index, Wp1, bp1, Ws1, Wn1, b1, Wp2, bp2, Ws2, Wn2, b2,
           Ws3, Wn3, b3):
    edge_flat = edge_index.reshape(2 * E)
    srcl, cnts = _partition(edge_flat)
    hp1 = _tc_pool_in(x, Wp1, bp1)
    agg1 = _segmax(hp1, srcl, cnts)
    h1, hp2 = _tc_mid(x, agg1, Ws1, Wn1, b1, Wp2, bp2)
    agg2 = _segmax(hp2, srcl, cnts)
    h2, h2s = _tc_h2(h1, agg2, Ws2, Wn2, b2)
    ssum, degs = _segsum(edge_flat, h2s)
    out = _tc_out(h2, ssum, degs, Ws3, Wn3, b3)
    return out
